# baseline (speedup 1.0000x reference)
import functools

import jax
import jax.numpy as jnp
from jax import lax
from jax.experimental import pallas as pl
from jax.experimental.pallas import tpu as pltpu
from jax.experimental.pallas import tpu_sc as plsc

_NW = 32
_L = 16
_V = 1000001
_D = 32
_BPW = 246
_LAST_B0 = 999808
_TAIL_B0 = 999936


def _wid():
    return lax.axis_index("s") * 2 + lax.axis_index("c")


def _conv_body(tt_hbm, tail_hbm, flat_hbm, tin, tout, sg0, sg1, sw0, sw1):
    w = _wid()
    lo = w * _BPW
    sg = (sg0, sg1)
    sw = (sw0, sw1)

    def b0_of(i):
        return jnp.minimum(i * 128, _LAST_B0)

    def issue_in(i, par):
        pltpu.async_copy(tt_hbm.at[:, pl.ds(b0_of(i), 128)],
                         tin.at[pl.ds(par * _D, _D)], sg[par])

    def wait_in(par):
        pltpu.make_async_copy(tt_hbm.at[:, pl.ds(0, 128)],
                              tin.at[pl.ds(par * _D, _D)], sg[par]).wait()

    def drain_out(par):
        pltpu.make_async_copy(flat_hbm.at[pl.ds(0, _D * 128)],
                              tout.at[pl.ds(par * _D * 128, _D * 128)],
                              sw[par]).wait()

    def transpose(par):
        iota = jax.lax.iota(jnp.int32, _L)

        def dstep(d4, carry):
            for dd in range(4):
                d = d4 * 4 + dd
                t = (iota + d) & (_L - 1)
                u = t * _D + iota
                for m in range(_D // _L):
                    cvec = iota + (par * _D + m * _L)
                    for g in range(128 // _L):
                        bvec = t + g * _L
                        v = plsc.load_gather(tin, [cvec, bvec])
                        sidx = u + (par * _D * 128 + g * _L * _D + m * _L)
                        plsc.store_scatter(tout, [sidx], v)
            return carry

        lax.fori_loop(0, _L // 4, dstep, 0)

    def issue_out(i, par):
        pltpu.async_copy(tout.at[pl.ds(par * _D * 128, _D * 128)],
                         flat_hbm.at[pl.ds(b0_of(i) * _D, _D * 128)],
                         sw[par])

    issue_in(lo, 0)
    issue_in(lo + 1, 1)

    def body(j, carry):
        for par in (0, 1):
            i = lo + 2 * j + par
            wait_in(par)

            @pl.when(j >= 1)
            def _():
                drain_out(par)

            transpose(par)
            issue_out(i, par)

            @pl.when(j < (_BPW // 2) - 1)
            def _():
                issue_in(i + 2, par)
        return carry

    lax.fori_loop(0, _BPW // 2, body, 0)
    drain_out(0)
    drain_out(1)

    @pl.when(w == 0)
    def _():
        pltpu.sync_copy(tail_hbm, tout.at[pl.ds(0, 65 * _D)])
        pltpu.sync_copy(tout.at[pl.ds(0, 65 * _D)],
                        flat_hbm.at[pl.ds(_TAIL_B0 * _D, 65 * _D)])


def _gather_body(n_s, xts_hbm, tab_hbm, out_hbm, xbuf, rows, rowsT,
                 sg0, sg1, sw0, sw1):
    w = _wid()
    sg = (sg0, sg1)
    sw = (sw0, sw1)
    pltpu.sync_copy(xts_hbm.at[:, pl.ds(w * 128, 128)], xbuf)

    def issue_g(s, par):
        pltpu.async_copy(tab_hbm.at[xbuf.at[s]], rows.at[par], sg[par])

    def wait_g(par):
        pltpu.make_async_copy(tab_hbm.at[pl.ds(0, 128)], rows.at[par],
                              sg[par]).wait()

    def drain_w(par):
        pltpu.make_async_copy(out_hbm.at[pl.ds(0, _D * 128)],
                              rowsT.at[pl.ds(par * _D * 128, _D * 128)],
                              sw[par]).wait()

    def transpose(par):
        iota = jax.lax.iota(jnp.int32, _L)

        def dstep(d4, carry):
            for dd in range(4):
                d = d4 * 4 + dd
                t = (iota + d) & (_L - 1)
                for m in range(_D // _L):
                    u = (iota + m * _L) * 128 + t
                    for g in range(128 // _L):
                        bvec = t + g * _L
                        cvec = iota + m * _L
                        v = plsc.load_gather(rows.at[par], [bvec, cvec])
                        sidx = u + (par * _D * 128 + g * _L)
                        plsc.store_scatter(rowsT, [sidx], v)
            return carry

        lax.fori_loop(0, _L // 4, dstep, 0)

    def issue_w(s, par):
        for cb in range(_D // 8):
            off = (((s * (_D // 8) + cb) * _NW) + w) * 1024
            pltpu.async_copy(
                rowsT.at[pl.ds(par * _D * 128 + cb * 1024, 1024)],
                out_hbm.at[pl.ds(off, 1024)], sw[par])

    issue_g(0, 0)
    issue_g(1, 1)

    def body(j, carry):
        for par in (0, 1):
            s = 2 * j + par
            wait_g(par)

            @pl.when(j >= 1)
            def _():
                drain_w(par)

            transpose(par)
            issue_w(s, par)

            @pl.when(j < (n_s // 2) - 1)
            def _():
                issue_g(s + 2, par)
        return carry

    lax.fori_loop(0, n_s // 2, body, 0)
    drain_w(0)
    drain_w(1)


def kernel(x, table):
    s0, s1 = x.shape
    v, d = table.shape
    xts = jnp.transpose(x).astype(jnp.int32)
    tt = jnp.transpose(table)

    mesh = plsc.VectorSubcoreMesh(core_axis_name="c", subcore_axis_name="s")

    conv = pl.kernel(
        _conv_body,
        mesh=mesh,
        out_type=jax.ShapeDtypeStruct((v * d,), jnp.float32),
        scratch_types=[
            pltpu.VMEM((2 * d, 128), jnp.float32),
            pltpu.VMEM((2 * d * 128,), jnp.float32),
            pltpu.SemaphoreType.DMA,
            pltpu.SemaphoreType.DMA,
            pltpu.SemaphoreType.DMA,
            pltpu.SemaphoreType.DMA,
        ],
        compiler_params=pltpu.CompilerParams(use_tc_tiling_on_sc=True,
                                             needs_layout_passes=False),
    )
    tail = lax.slice(table, (_TAIL_B0, 0), (v, d)).reshape(-1)
    flat_tab = conv(tt, tail)
    tab = flat_tab.reshape(v, d)

    gather = pl.kernel(
        functools.partial(_gather_body, s1),
        mesh=mesh,
        out_type=jax.ShapeDtypeStruct((s0 * s1 * d,), jnp.float32),
        scratch_types=[
            pltpu.VMEM((s1, 128), jnp.int32),
            pltpu.VMEM((2, 128, d), jnp.float32),
            pltpu.VMEM((2 * d * 128,), jnp.float32),
            pltpu.SemaphoreType.DMA,
            pltpu.SemaphoreType.DMA,
            pltpu.SemaphoreType.DMA,
            pltpu.SemaphoreType.DMA,
        ],
        compiler_params=pltpu.CompilerParams(use_tc_tiling_on_sc=False,
                                             needs_layout_passes=False),
    )
    flat = gather(xts, tab)
    out = flat.reshape(s1, d // 8, s0 // 128, 8, 128)
    out = out.transpose(2, 4, 0, 1, 3).reshape(s0, s1, d)
    return out


# speedup vs baseline: 1.0240x; 1.0240x over previous
import functools

import jax
import jax.numpy as jnp
from jax import lax
from jax.experimental import pallas as pl
from jax.experimental.pallas import tpu as pltpu
from jax.experimental.pallas import tpu_sc as plsc

_NW = 32
_L = 16
_V = 1000001
_D = 32
_BPW = 248
_NBUF = 4
_LAST_B0 = 999808
_TAIL_B0 = 999936


def _wid():
    return lax.axis_index("s") * 2 + lax.axis_index("c")


def _conv_body(tt_hbm, tail_hbm, flat_hbm, tin, tout,
               sg0, sg1, sg2, sg3, sw0, sw1, sw2, sw3):
    w = _wid()
    lo = w * _BPW
    sg = (sg0, sg1, sg2, sg3)
    sw = (sw0, sw1, sw2, sw3)

    def b0_of(i):
        return jnp.minimum(i * 128, _LAST_B0)

    def issue_in(i, par):
        pltpu.async_copy(tt_hbm.at[:, pl.ds(b0_of(i), 128)],
                         tin.at[pl.ds(par * _D, _D)], sg[par])

    def wait_in(par):
        pltpu.make_async_copy(tt_hbm.at[:, pl.ds(0, 128)],
                              tin.at[pl.ds(par * _D, _D)], sg[par]).wait()

    def drain_out(par):
        pltpu.make_async_copy(flat_hbm.at[pl.ds(0, _D * 128)],
                              tout.at[pl.ds(par * _D * 128, _D * 128)],
                              sw[par]).wait()

    def transpose(par):
        iota = jax.lax.iota(jnp.int32, _L)

        def dstep(d, carry):
            t = (iota + d) & (_L - 1)
            u = t * _D + iota
            for m in range(_D // _L):
                cvec = iota + (par * _D + m * _L)
                for g in range(128 // _L):
                    bvec = t + g * _L
                    v = plsc.load_gather(tin, [cvec, bvec])
                    sidx = u + (par * _D * 128 + g * _L * _D + m * _L)
                    plsc.store_scatter(tout, [sidx], v)
            return carry

        lax.fori_loop(0, _L, dstep, 0)

    def issue_out(i, par):
        pltpu.async_copy(tout.at[pl.ds(par * _D * 128, _D * 128)],
                         flat_hbm.at[pl.ds(b0_of(i) * _D, _D * 128)],
                         sw[par])

    for q in range(_NBUF):
        issue_in(lo + q, q)

    def body(j, carry):
        for par in range(_NBUF):
            i = lo + _NBUF * j + par
            wait_in(par)

            @pl.when(j >= 1)
            def _():
                drain_out(par)

            transpose(par)
            issue_out(i, par)

            @pl.when(j < (_BPW // _NBUF) - 1)
            def _():
                issue_in(i + _NBUF, par)
        return carry

    lax.fori_loop(0, _BPW // _NBUF, body, 0)
    for q in range(_NBUF):
        drain_out(q)

    @pl.when(w == 0)
    def _():
        pltpu.sync_copy(tail_hbm, tout.at[pl.ds(0, 65 * _D)])
        pltpu.sync_copy(tout.at[pl.ds(0, 65 * _D)],
                        flat_hbm.at[pl.ds(_TAIL_B0 * _D, 65 * _D)])


def _gather_body(n_s, xts_hbm, tab_hbm, out_hbm, xbuf, rows, rowsT,
                 sg0, sg1, sw0, sw1):
    w = _wid()
    sg = (sg0, sg1)
    sw = (sw0, sw1)
    pltpu.sync_copy(xts_hbm.at[:, pl.ds(w * 128, 128)], xbuf)

    def issue_g(s, par):
        pltpu.async_copy(tab_hbm.at[xbuf.at[s]], rows.at[par], sg[par])

    def wait_g(par):
        pltpu.make_async_copy(tab_hbm.at[pl.ds(0, 128)], rows.at[par],
                              sg[par]).wait()

    def drain_w(par):
        pltpu.make_async_copy(out_hbm.at[pl.ds(0, _D * 128)],
                              rowsT.at[pl.ds(par * _D * 128, _D * 128)],
                              sw[par]).wait()

    def transpose(par):
        iota = jax.lax.iota(jnp.int32, _L)

        def dstep(d, carry):
            t = (iota + d) & (_L - 1)
            for m in range(_D // _L):
                u = (iota + m * _L) * 128 + t
                for g in range(128 // _L):
                    bvec = t + g * _L
                    cvec = iota + m * _L
                    v = plsc.load_gather(rows.at[par], [bvec, cvec])
                    sidx = u + (par * _D * 128 + g * _L)
                    plsc.store_scatter(rowsT, [sidx], v)
            return carry

        lax.fori_loop(0, _L, dstep, 0)

    def issue_w(s, par):
        for cb in range(_D // 8):
            off = (((s * (_D // 8) + cb) * _NW) + w) * 1024
            pltpu.async_copy(
                rowsT.at[pl.ds(par * _D * 128 + cb * 1024, 1024)],
                out_hbm.at[pl.ds(off, 1024)], sw[par])

    issue_g(0, 0)
    issue_g(1, 1)

    def body(j, carry):
        for par in (0, 1):
            s = 2 * j + par
            wait_g(par)

            @pl.when(j >= 1)
            def _():
                drain_w(par)

            transpose(par)
            issue_w(s, par)

            @pl.when(j < (n_s // 2) - 1)
            def _():
                issue_g(s + 2, par)
        return carry

    lax.fori_loop(0, n_s // 2, body, 0)
    drain_w(0)
    drain_w(1)


def kernel(x, table):
    s0, s1 = x.shape
    v, d = table.shape
    xts = jnp.transpose(x).astype(jnp.int32)
    tt = jnp.transpose(table)

    mesh = plsc.VectorSubcoreMesh(core_axis_name="c", subcore_axis_name="s")

    conv = pl.kernel(
        _conv_body,
        mesh=mesh,
        out_type=jax.ShapeDtypeStruct((v * d,), jnp.float32),
        scratch_types=[
            pltpu.VMEM((_NBUF * d, 128), jnp.float32),
            pltpu.VMEM((_NBUF * d * 128,), jnp.float32),
        ] + [pltpu.SemaphoreType.DMA] * (2 * _NBUF),
        compiler_params=pltpu.CompilerParams(use_tc_tiling_on_sc=True,
                                             needs_layout_passes=False),
    )
    tail = lax.slice(table, (_TAIL_B0, 0), (v, d)).reshape(-1)
    flat_tab = conv(tt, tail)
    tab = flat_tab.reshape(v, d)

    gather = pl.kernel(
        functools.partial(_gather_body, s1),
        mesh=mesh,
        out_type=jax.ShapeDtypeStruct((s0 * s1 * d,), jnp.float32),
        scratch_types=[
            pltpu.VMEM((s1, 128), jnp.int32),
            pltpu.VMEM((2, 128, d), jnp.float32),
            pltpu.VMEM((2 * d * 128,), jnp.float32),
            pltpu.SemaphoreType.DMA,
            pltpu.SemaphoreType.DMA,
            pltpu.SemaphoreType.DMA,
            pltpu.SemaphoreType.DMA,
        ],
        compiler_params=pltpu.CompilerParams(use_tc_tiling_on_sc=False,
                                             needs_layout_passes=False),
    )
    flat = gather(xts, tab)
    out = flat.reshape(s1, d // 8, s0 // 128, 8, 128)
    out = out.transpose(2, 4, 0, 1, 3).reshape(s0, s1, d)
    return out


# speedup vs baseline: 1.0283x; 1.0043x over previous
import functools

import jax
import jax.numpy as jnp
from jax import lax
from jax.experimental import pallas as pl
from jax.experimental.pallas import tpu as pltpu
from jax.experimental.pallas import tpu_sc as plsc

_NW = 32
_L = 16
_V = 1000001
_D = 32
_BPW = 248
_NBUF = 4
_LAST_B0 = 999808
_TAIL_B0 = 999936


def _wid():
    return lax.axis_index("s") * 2 + lax.axis_index("c")


def _conv_body(tt_hbm, tail_hbm, flat_hbm, tin, tout,
               sg0, sg1, sg2, sg3, sw0, sw1, sw2, sw3):
    w = _wid()
    lo = w * _BPW
    sg = (sg0, sg1, sg2, sg3)
    sw = (sw0, sw1, sw2, sw3)

    def b0_of(i):
        return jnp.minimum(i * 128, _LAST_B0)

    def issue_in(i, par):
        pltpu.async_copy(tt_hbm.at[:, pl.ds(b0_of(i), 128)],
                         tin.at[pl.ds(par * _D, _D)], sg[par])

    def wait_in(par):
        pltpu.make_async_copy(tt_hbm.at[:, pl.ds(0, 128)],
                              tin.at[pl.ds(par * _D, _D)], sg[par]).wait()

    def drain_out(par):
        pltpu.make_async_copy(flat_hbm.at[pl.ds(0, _D * 128)],
                              tout.at[pl.ds(par * _D * 128, _D * 128)],
                              sw[par]).wait()

    def transpose(par):
        iota = jax.lax.iota(jnp.int32, _L)

        def dstep(d2, carry):
            for dd in range(2):
                d = d2 * 2 + dd
                t = (iota + d) & (_L - 1)
                u = t * _D + iota
                for m in range(_D // _L):
                    cvec = iota + (par * _D + m * _L)
                    for g in range(128 // _L):
                        bvec = t + g * _L
                        v = plsc.load_gather(tin, [cvec, bvec])
                        sidx = u + (par * _D * 128 + g * _L * _D + m * _L)
                        plsc.store_scatter(tout, [sidx], v)
            return carry

        lax.fori_loop(0, _L // 2, dstep, 0)

    def issue_out(i, par):
        pltpu.async_copy(tout.at[pl.ds(par * _D * 128, _D * 128)],
                         flat_hbm.at[pl.ds(b0_of(i) * _D, _D * 128)],
                         sw[par])

    for q in range(_NBUF):
        issue_in(lo + q, q)

    def body(j, carry):
        for par in range(_NBUF):
            i = lo + _NBUF * j + par
            wait_in(par)

            @pl.when(j >= 1)
            def _():
                drain_out(par)

            transpose(par)
            issue_out(i, par)

            @pl.when(j < (_BPW // _NBUF) - 1)
            def _():
                issue_in(i + _NBUF, par)
        return carry

    lax.fori_loop(0, _BPW // _NBUF, body, 0)
    for q in range(_NBUF):
        drain_out(q)

    @pl.when(w == 0)
    def _():
        pltpu.sync_copy(tail_hbm, tout.at[pl.ds(0, 65 * _D)])
        pltpu.sync_copy(tout.at[pl.ds(0, 65 * _D)],
                        flat_hbm.at[pl.ds(_TAIL_B0 * _D, 65 * _D)])


def _gather_body(n_s, xts_hbm, tab_hbm, out_hbm, xbuf, rows, rowsT,
                 sg0, sg1, sw0, sw1):
    w = _wid()
    sg = (sg0, sg1)
    sw = (sw0, sw1)
    pltpu.sync_copy(xts_hbm.at[:, pl.ds(w * 128, 128)], xbuf)

    def issue_g(s, par):
        pltpu.async_copy(tab_hbm.at[xbuf.at[s]], rows.at[par], sg[par])

    def wait_g(par):
        pltpu.make_async_copy(tab_hbm.at[pl.ds(0, 128)], rows.at[par],
                              sg[par]).wait()

    def drain_w(par):
        pltpu.make_async_copy(out_hbm.at[pl.ds(0, _D * 128)],
                              rowsT.at[pl.ds(par * _D * 128, _D * 128)],
                              sw[par]).wait()

    def transpose(par):
        iota = jax.lax.iota(jnp.int32, _L)

        def dstep(d, carry):
            t = (iota + d) & (_L - 1)
            for m in range(_D // _L):
                u = (iota + m * _L) * 128 + t
                for g in range(128 // _L):
                    bvec = t + g * _L
                    cvec = iota + m * _L
                    v = plsc.load_gather(rows.at[par], [bvec, cvec])
                    sidx = u + (par * _D * 128 + g * _L)
                    plsc.store_scatter(rowsT, [sidx], v)
            return carry

        lax.fori_loop(0, _L, dstep, 0)

    def issue_w(s, par):
        for cb in range(_D // 8):
            off = (((s * (_D // 8) + cb) * _NW) + w) * 1024
            pltpu.async_copy(
                rowsT.at[pl.ds(par * _D * 128 + cb * 1024, 1024)],
                out_hbm.at[pl.ds(off, 1024)], sw[par])

    issue_g(0, 0)
    issue_g(1, 1)

    def body(j, carry):
        for par in (0, 1):
            s = 2 * j + par
            wait_g(par)

            @pl.when(j >= 1)
            def _():
                drain_w(par)

            transpose(par)
            issue_w(s, par)

            @pl.when(j < (n_s // 2) - 1)
            def _():
                issue_g(s + 2, par)
        return carry

    lax.fori_loop(0, n_s // 2, body, 0)
    drain_w(0)
    drain_w(1)


def kernel(x, table):
    s0, s1 = x.shape
    v, d = table.shape
    xts = jnp.transpose(x).astype(jnp.int32)
    tt = jnp.transpose(table)

    mesh = plsc.VectorSubcoreMesh(core_axis_name="c", subcore_axis_name="s")

    conv = pl.kernel(
        _conv_body,
        mesh=mesh,
        out_type=jax.ShapeDtypeStruct((v * d,), jnp.float32),
        scratch_types=[
            pltpu.VMEM((_NBUF * d, 128), jnp.float32),
            pltpu.VMEM((_NBUF * d * 128,), jnp.float32),
        ] + [pltpu.SemaphoreType.DMA] * (2 * _NBUF),
        compiler_params=pltpu.CompilerParams(use_tc_tiling_on_sc=True,
                                             needs_layout_passes=False),
    )
    tail = lax.slice(table, (_TAIL_B0, 0), (v, d)).reshape(-1)
    flat_tab = conv(tt, tail)
    tab = flat_tab.reshape(v, d)

    gather = pl.kernel(
        functools.partial(_gather_body, s1),
        mesh=mesh,
        out_type=jax.ShapeDtypeStruct((s0 * s1 * d,), jnp.float32),
        scratch_types=[
            pltpu.VMEM((s1, 128), jnp.int32),
            pltpu.VMEM((2, 128, d), jnp.float32),
            pltpu.VMEM((2 * d * 128,), jnp.float32),
            pltpu.SemaphoreType.DMA,
            pltpu.SemaphoreType.DMA,
            pltpu.SemaphoreType.DMA,
            pltpu.SemaphoreType.DMA,
        ],
        compiler_params=pltpu.CompilerParams(use_tc_tiling_on_sc=False,
                                             needs_layout_passes=False),
    )
    flat = gather(xts, tab)
    out = flat.reshape(s1, d // 8, s0 // 128, 8, 128)
    out = out.transpose(2, 4, 0, 1, 3).reshape(s0, s1, d)
    return out
